# Initial kernel scaffold; baseline (speedup 1.0000x reference)
import jax
import jax.numpy as jnp
from jax.experimental import pallas as pl


def kernel(h_n, h_e, edge_index, Wq_w, Wq_b, Wkv_w, Wkv_b, Wo_w, Wo_b, ln1_g, ln1_b, W1, b1, W2, b2, ln2_g, ln2_b):
    raise NotImplementedError("write your pallas kernel here")


# speedup vs baseline: 21.0100x; 21.0100x over previous
import functools
import math

import jax
import jax.numpy as jnp
from jax import lax
from jax.experimental import pallas as pl
from jax.experimental.pallas import tpu as pltpu
from jax.experimental.pallas import tpu_sc as plsc

N = 10000
E = 320000
D = 128
NH = 4
HD = 32

CH = 125
NCHUNK = E // CH
NW = 32
PERW = NCHUNK // NW
NPAD = 10240
TROWS = NPAD // 16

_INV_SQRT_HD = 1.0 / math.sqrt(HD)
_INV_SQRT2 = 1.0 / math.sqrt(2.0)


def _node_proj_body(hn_ref, w_ref, b_ref, qn_ref, kn_ref, vn_ref):
    o = jnp.dot(hn_ref[...], w_ref[...], preferred_element_type=jnp.float32)
    o = o + b_ref[...]
    qn_ref[...] = o[:, :D]
    kn_ref[...] = o[:, D:2 * D]
    vn_ref[...] = o[:, 2 * D:]


def _node_proj(h_n, w_node, b_node):
    blk = 2000
    return pl.pallas_call(
        _node_proj_body,
        grid=(N // blk,),
        in_specs=[
            pl.BlockSpec((blk, D), lambda i: (i, 0)),
            pl.BlockSpec((D, 3 * D), lambda i: (0, 0)),
            pl.BlockSpec((1, 3 * D), lambda i: (0, 0)),
        ],
        out_specs=[
            pl.BlockSpec((blk, D), lambda i: (i, 0)),
            pl.BlockSpec((blk, D), lambda i: (i, 0)),
            pl.BlockSpec((blk, D), lambda i: (i, 0)),
        ],
        out_shape=[
            jax.ShapeDtypeStruct((N, D), jnp.float32),
            jax.ShapeDtypeStruct((N, D), jnp.float32),
            jax.ShapeDtypeStruct((N, D), jnp.float32),
        ],
    )(h_n, w_node, b_node)


def _sc_gather(qn, kn, vn, src3, dst3):
    mesh = plsc.VectorSubcoreMesh(core_axis_name="c", subcore_axis_name="s")

    @functools.partial(
        pl.kernel,
        out_type=(
            jax.ShapeDtypeStruct((NCHUNK, CH, D), jnp.float32),
            jax.ShapeDtypeStruct((NCHUNK, CH, D), jnp.float32),
            jax.ShapeDtypeStruct((NCHUNK, CH, D), jnp.float32),
        ),
        mesh=mesh,
        scratch_types=[
            pltpu.VMEM((PERW, CH), jnp.int32),
            pltpu.VMEM((PERW, CH), jnp.int32),
            pltpu.VMEM((CH, D), jnp.float32),
            pltpu.VMEM((CH, D), jnp.float32),
            pltpu.VMEM((CH, D), jnp.float32),
            pltpu.SemaphoreType.DMA,
            pltpu.SemaphoreType.DMA,
            pltpu.SemaphoreType.DMA,
        ],
    )
    def k(qn_hbm, kn_hbm, vn_hbm, src_hbm, dst_hbm, qns_hbm, kns_hbm, vns_hbm,
          src_v, dst_v, qbuf, kbuf, vbuf, sem1, sem2, sem3):
        c = lax.axis_index("c")
        s = lax.axis_index("s")
        wid = s * 2 + c
        pltpu.sync_copy(src_hbm.at[pl.ds(wid * PERW, PERW)], src_v)
        pltpu.sync_copy(dst_hbm.at[pl.ds(wid * PERW, PERW)], dst_v)

        def body(j, carry):
            chunk = wid * PERW + j
            cp1 = pltpu.async_copy(qn_hbm.at[src_v.at[j]], qbuf, sem1)
            cp2 = pltpu.async_copy(kn_hbm.at[dst_v.at[j]], kbuf, sem2)
            cp3 = pltpu.async_copy(vn_hbm.at[dst_v.at[j]], vbuf, sem3)
            cp1.wait()
            cp2.wait()
            cp3.wait()
            pltpu.sync_copy(qbuf, qns_hbm.at[chunk])
            pltpu.sync_copy(kbuf, kns_hbm.at[chunk])
            pltpu.sync_copy(vbuf, vns_hbm.at[chunk])
            return carry

        lax.fori_loop(0, PERW, body, 0)

    return k(qn, kn, vn, src3, dst3)


def _edge_body(he_ref, qns_ref, kns_ref, vns_ref, we_ref, m16_ref, ex_ref, pv_ref, p_ref):
    he = he_ref[...]
    qkve = jnp.dot(he, we_ref[...], preferred_element_type=jnp.float32)
    q = qkve[:, :D] + qns_ref[...]
    k = qkve[:, D:2 * D] + kns_ref[...]
    v = qkve[:, 2 * D:] + vns_ref[...]
    s16 = jnp.dot(q * k, m16_ref[...], preferred_element_type=jnp.float32)
    s16 = s16 * _INV_SQRT_HD
    col = lax.broadcasted_iota(jnp.int32, s16.shape, 1)
    p16 = jnp.where(col < NH, jnp.exp(s16), 0.0)
    pv_ref[...] = v * jnp.dot(p16, ex_ref[...], preferred_element_type=jnp.float32)
    p_ref[...] = p16


def _edge_stage(h_e, qns, kns, vns, w_edge, m16, ex16):
    blk = 2560
    return pl.pallas_call(
        _edge_body,
        grid=(E // blk,),
        in_specs=[
            pl.BlockSpec((blk, D), lambda i: (i, 0)),
            pl.BlockSpec((blk, D), lambda i: (i, 0)),
            pl.BlockSpec((blk, D), lambda i: (i, 0)),
            pl.BlockSpec((blk, D), lambda i: (i, 0)),
            pl.BlockSpec((D, 3 * D), lambda i: (0, 0)),
            pl.BlockSpec((D, 16), lambda i: (0, 0)),
            pl.BlockSpec((16, D), lambda i: (0, 0)),
        ],
        out_specs=[
            pl.BlockSpec((blk, D), lambda i: (i, 0)),
            pl.BlockSpec((blk, 16), lambda i: (i, 0)),
        ],
        out_shape=[
            jax.ShapeDtypeStruct((E, D), jnp.float32),
            jax.ShapeDtypeStruct((E, 16), jnp.float32),
        ],
    )(h_e, qns, kns, vns, w_edge, m16, ex16)


PERC = NCHUNK // 16


def _sc_scatter(pv3, p3, src3, z128):
    mesh = plsc.VectorSubcoreMesh(core_axis_name="c", subcore_axis_name="s")

    @functools.partial(
        pl.kernel,
        out_type=jax.ShapeDtypeStruct((2, NPAD, D), jnp.float32),
        mesh=mesh,
        scratch_types=[
            pltpu.VMEM_SHARED((NPAD, D), jnp.float32),
            pltpu.VMEM((CH, D), jnp.float32),
            pltpu.VMEM((CH, 16), jnp.float32),
            pltpu.VMEM((CH,), jnp.int32),
        ],
    )
    def k(pv_hbm, p_hbm, src_hbm, z_hbm, outacc_hbm, acc_sp, pvbuf, pbuf, idxrow):
        c = lax.axis_index("c")
        s = lax.axis_index("s")
        pltpu.sync_copy(z_hbm, acc_sp.at[pl.ds(s * TROWS, TROWS)])
        plsc.subcore_barrier()

        @pl.when(c == 0)
        def _():
            def body(j, carry):
                chunk = s * PERC + j
                pltpu.sync_copy(src_hbm.at[chunk], idxrow)
                pltpu.sync_copy(pv_hbm.at[chunk], pvbuf)
                pltpu.sync_copy(pvbuf, acc_sp.at[idxrow], add=True)
                return carry

            lax.fori_loop(0, PERC, body, 0)

        @pl.when(c == 1)
        def _():
            def zb(r, carry):
                def zg(g, carry2):
                    pvbuf[r, pl.ds(g * 16, 16)] = jnp.zeros((16,), jnp.float32)
                    return carry2
                return lax.fori_loop(0, 8, zg, carry)

            lax.fori_loop(0, CH, zb, 0)

            def body(j, carry):
                chunk = s * PERC + j
                pltpu.sync_copy(src_hbm.at[chunk], idxrow)
                pltpu.sync_copy(p_hbm.at[chunk], pbuf)

                def cp(r, carry2):
                    pvbuf[r, pl.ds(0, 16)] = pbuf[r, ...]
                    return carry2

                lax.fori_loop(0, CH, cp, carry)
                pltpu.sync_copy(pvbuf, acc_sp.at[idxrow], add=True)
                return carry

            lax.fori_loop(0, PERC, body, 0)

        plsc.subcore_barrier()
        pltpu.sync_copy(acc_sp.at[pl.ds(s * TROWS, TROWS)],
                        outacc_hbm.at[c, pl.ds(s * TROWS, TROWS)])

    return k(pv3, p3, src3, z128)


def _ln(x, g, b):
    m = jnp.mean(x, axis=-1, keepdims=True)
    v = jnp.mean((x - m) ** 2, axis=-1, keepdims=True)
    return (x - m) * lax.rsqrt(v + 1e-5) * g + b


def _final_body(accp_ref, hn_ref, ex_ref, wo_ref, bo_ref, g1_ref, bl1_ref,
                w1_ref, b1_ref, w2_ref, b2_ref, g2_ref, bl2_ref, out_ref):
    acc = accp_ref[0]
    s16 = accp_ref[1][:, :16]
    den = jnp.dot(s16, ex_ref[...], preferred_element_type=jnp.float32)
    den = jnp.where(den == 0.0, 1.0, den)
    attn = acc / den
    o = jnp.dot(attn, wo_ref[...], preferred_element_type=jnp.float32) + bo_ref[...]
    x = hn_ref[...] + o
    h1 = _ln(x, g1_ref[...], bl1_ref[...])
    f = jnp.dot(h1, w1_ref[...], preferred_element_type=jnp.float32) + b1_ref[...]
    f = 0.5 * f * (1.0 + lax.erf(f * _INV_SQRT2))
    f = jnp.dot(f, w2_ref[...], preferred_element_type=jnp.float32) + b2_ref[...]
    out_ref[...] = _ln(h1 + f, g2_ref[...], bl2_ref[...])


def _final_stage(outacc, h_n, ex16, wo_t, bo, g1, bl1, w1_t, b1_, w2_t, b2_, g2, bl2):
    blk = 2000
    full = lambda shape: pl.BlockSpec(shape, lambda i: tuple(0 for _ in shape))
    return pl.pallas_call(
        _final_body,
        grid=(N // blk,),
        in_specs=[
            pl.BlockSpec((2, blk, D), lambda i: (0, i, 0)),
            pl.BlockSpec((blk, D), lambda i: (i, 0)),
            full((16, D)),
            full((D, D)),
            full((1, D)),
            full((1, D)),
            full((1, D)),
            full((D, 4 * D)),
            full((1, 4 * D)),
            full((4 * D, D)),
            full((1, D)),
            full((1, D)),
            full((1, D)),
        ],
        out_specs=pl.BlockSpec((blk, D), lambda i: (i, 0)),
        out_shape=jax.ShapeDtypeStruct((N, D), jnp.float32),
    )(outacc, h_n, ex16, wo_t, bo, g1, bl1, w1_t, b1_, w2_t, b2_, g2, bl2)


def kernel(h_n, h_e, edge_index, Wq_w, Wq_b, Wkv_w, Wkv_b, Wo_w, Wo_b,
           ln1_g, ln1_b, W1, b1, W2, b2, ln2_g, ln2_b):
    f32 = jnp.float32
    src3 = edge_index[0].reshape(NCHUNK, CH)
    dst3 = edge_index[1].reshape(NCHUNK, CH)

    w_node = jnp.concatenate([Wq_w[:, :D].T, Wkv_w[:, :D].T], axis=1)
    b_node = jnp.concatenate([Wq_b, Wkv_b])[None, :]
    w_edge = jnp.concatenate([Wq_w[:, D:].T, Wkv_w[:, D:].T], axis=1)

    hd_ids = jnp.arange(D) // HD
    m16 = (hd_ids[:, None] == jnp.arange(16)[None, :]).astype(f32)
    ex16 = m16.T

    z128 = jnp.zeros((TROWS, D), f32)

    qn, kn, vn = _node_proj(h_n, w_node, b_node)
    qns, kns, vns = _sc_gather(qn, kn, vn, src3, dst3)
    pv, p = _edge_stage(h_e, qns.reshape(E, D), kns.reshape(E, D),
                        vns.reshape(E, D), w_edge, m16, ex16)
    outacc = _sc_scatter(pv.reshape(NCHUNK, CH, D),
                         p.reshape(NCHUNK, CH, 16), src3, z128)
    return _final_stage(outacc, h_n, ex16,
                        Wo_w.T, Wo_b[None, :], ln1_g[None, :], ln1_b[None, :],
                        W1.T, b1[None, :], W2.T, b2[None, :],
                        ln2_g[None, :], ln2_b[None, :])


# speedup vs baseline: 37.3067x; 1.7757x over previous
import functools
import math

import jax
import jax.numpy as jnp
from jax import lax
from jax.experimental import pallas as pl
from jax.experimental.pallas import tpu as pltpu
from jax.experimental.pallas import tpu_sc as plsc

N = 10000
E = 320000
D = 128
NH = 4
HD = 32

CH = 80
NCHUNK = E // CH
NW = 32
PERW = NCHUNK // NW
PERC = NCHUNK // 16
CPG = CH // 8
NPAD = 10240
TROWS = NPAD // 16

_INV_SQRT_HD = 1.0 / math.sqrt(HD)
_INV_SQRT2 = 1.0 / math.sqrt(2.0)


def _node_proj_body(hn_ref, w_ref, b_ref, qn_ref, kn_ref, vn_ref):
    o = jnp.dot(hn_ref[...], w_ref[...], preferred_element_type=jnp.float32)
    o = o + b_ref[...]
    qn_ref[...] = o[:, :D]
    kn_ref[...] = o[:, D:2 * D]
    vn_ref[...] = o[:, 2 * D:]


def _node_proj(h_n, w_node, b_node):
    blk = 2000
    return pl.pallas_call(
        _node_proj_body,
        grid=(N // blk,),
        in_specs=[
            pl.BlockSpec((blk, D), lambda i: (i, 0)),
            pl.BlockSpec((D, 3 * D), lambda i: (0, 0)),
            pl.BlockSpec((1, 3 * D), lambda i: (0, 0)),
        ],
        out_specs=[
            pl.BlockSpec((blk, D), lambda i: (i, 0)),
            pl.BlockSpec((blk, D), lambda i: (i, 0)),
            pl.BlockSpec((blk, D), lambda i: (i, 0)),
        ],
        out_shape=[
            jax.ShapeDtypeStruct((N, D), jnp.float32),
            jax.ShapeDtypeStruct((N, D), jnp.float32),
            jax.ShapeDtypeStruct((N, D), jnp.float32),
        ],
    )(h_n, w_node, b_node)


def _sc_gather(qn, kn, vn, src3, dst3):
    mesh = plsc.VectorSubcoreMesh(core_axis_name="c", subcore_axis_name="s")

    @functools.partial(
        pl.kernel,
        out_type=(
            jax.ShapeDtypeStruct((NCHUNK, CH, D), jnp.float32),
            jax.ShapeDtypeStruct((NCHUNK, CH, D), jnp.float32),
            jax.ShapeDtypeStruct((NCHUNK, CH, D), jnp.float32),
        ),
        mesh=mesh,
        scratch_types=(
            [pltpu.VMEM((PERW, CH), jnp.int32)] * 2
            + [pltpu.VMEM((CH, D), jnp.float32)] * 6
            + [pltpu.SemaphoreType.DMA] * 12
        ),
    )
    def k(qn_hbm, kn_hbm, vn_hbm, src_hbm, dst_hbm, qns_hbm, kns_hbm, vns_hbm,
          src_v, dst_v, qbA, kbA, vbA, qbB, kbB, vbB,
          gqA, gkA, gvA, gqB, gkB, gvB, wqA, wkA, wvA, wqB, wkB, wvB):
        c = lax.axis_index("c")
        s = lax.axis_index("s")
        wid = s * 2 + c
        base = wid * PERW
        pltpu.sync_copy(src_hbm.at[wid], src_v)
        pltpu.sync_copy(dst_hbm.at[wid], dst_v)

        A = (qbA, kbA, vbA, gqA, gkA, gvA, wqA, wkA, wvA)
        B = (qbB, kbB, vbB, gqB, gkB, gvB, wqB, wkB, wvB)

        def issue_g(j, bufs):
            qb, kb, vb, gq, gk, gv = bufs[:6]
            pltpu.async_copy(qn_hbm.at[src_v.at[j]], qb, gq)
            pltpu.async_copy(kn_hbm.at[dst_v.at[j]], kb, gk)
            pltpu.async_copy(vn_hbm.at[dst_v.at[j]], vb, gv)

        def wait_g(j, bufs):
            qb, kb, vb, gq, gk, gv = bufs[:6]
            pltpu.make_async_copy(qn_hbm.at[src_v.at[j]], qb, gq).wait()
            pltpu.make_async_copy(kn_hbm.at[dst_v.at[j]], kb, gk).wait()
            pltpu.make_async_copy(vn_hbm.at[dst_v.at[j]], vb, gv).wait()

        def issue_w(j, bufs):
            qb, kb, vb = bufs[:3]
            wq, wk, wv = bufs[6:]
            chunk = base + j
            pltpu.async_copy(qb, qns_hbm.at[chunk], wq)
            pltpu.async_copy(kb, kns_hbm.at[chunk], wk)
            pltpu.async_copy(vb, vns_hbm.at[chunk], wv)

        def wait_w(j, bufs):
            qb, kb, vb = bufs[:3]
            wq, wk, wv = bufs[6:]
            chunk = base + j
            pltpu.make_async_copy(qb, qns_hbm.at[chunk], wq).wait()
            pltpu.make_async_copy(kb, kns_hbm.at[chunk], wk).wait()
            pltpu.make_async_copy(vb, vns_hbm.at[chunk], wv).wait()

        def step(j, cur, nxt):
            @pl.when(j + 1 < PERW)
            def _():
                @pl.when(j >= 1)
                def _():
                    wait_w(j - 1, nxt)

                issue_g(j + 1, nxt)

            wait_g(j, cur)
            issue_w(j, cur)

        issue_g(0, A)

        def body(j, carry):
            @pl.when(lax.rem(j, 2) == 0)
            def _():
                step(j, A, B)

            @pl.when(lax.rem(j, 2) == 1)
            def _():
                step(j, B, A)

            return carry

        lax.fori_loop(0, PERW, body, 0)
        wait_w(PERW - 1, A if (PERW - 1) % 2 == 0 else B)
        wait_w(PERW - 2, A if (PERW - 2) % 2 == 0 else B)

    return k(qn, kn, vn, src3, dst3)


def _edge_body(he_ref, qns_ref, kns_ref, vns_ref, we_ref, m16_ref, ex_ref, pv_ref, p_ref):
    he = he_ref[...]
    qkve = jnp.dot(he, we_ref[...], preferred_element_type=jnp.float32)
    q = qkve[:, :D] + qns_ref[...]
    k = qkve[:, D:2 * D] + kns_ref[...]
    v = qkve[:, 2 * D:] + vns_ref[...]
    s16 = jnp.dot(q * k, m16_ref[...], preferred_element_type=jnp.float32)
    s16 = s16 * _INV_SQRT_HD
    col = lax.broadcasted_iota(jnp.int32, s16.shape, 1)
    p16 = jnp.where(col < NH, jnp.exp(s16), 0.0)
    pv_ref[...] = v * jnp.dot(p16, ex_ref[...], preferred_element_type=jnp.float32)
    p_ref[...] = p16


def _edge_stage(h_e, qns, kns, vns, w_edge, m16, ex16):
    blk = 2560
    return pl.pallas_call(
        _edge_body,
        grid=(E // blk,),
        in_specs=[
            pl.BlockSpec((blk, D), lambda i: (i, 0)),
            pl.BlockSpec((blk, D), lambda i: (i, 0)),
            pl.BlockSpec((blk, D), lambda i: (i, 0)),
            pl.BlockSpec((blk, D), lambda i: (i, 0)),
            pl.BlockSpec((D, 3 * D), lambda i: (0, 0)),
            pl.BlockSpec((D, 16), lambda i: (0, 0)),
            pl.BlockSpec((16, D), lambda i: (0, 0)),
        ],
        out_specs=[
            pl.BlockSpec((blk, D), lambda i: (i, 0)),
            pl.BlockSpec((blk, 16), lambda i: (i, 0)),
        ],
        out_shape=[
            jax.ShapeDtypeStruct((E, D), jnp.float32),
            jax.ShapeDtypeStruct((E, 16), jnp.float32),
        ],
    )(h_e, qns, kns, vns, w_edge, m16, ex16)


def _sc_scatter(pv3, p3, src3, z128):
    mesh = plsc.VectorSubcoreMesh(core_axis_name="c", subcore_axis_name="s")

    @functools.partial(
        pl.kernel,
        out_type=jax.ShapeDtypeStruct((2, NPAD, D), jnp.float32),
        mesh=mesh,
        scratch_types=(
            [pltpu.VMEM_SHARED((NPAD, D), jnp.float32)]
            + [pltpu.VMEM((CH, D), jnp.float32)] * 2
            + [pltpu.VMEM((CPG, 128), jnp.float32)] * 2
            + [pltpu.VMEM((CH,), jnp.int32)] * 2
            + [pltpu.SemaphoreType.DMA] * 6
        ),
    )
    def k(pv_hbm, p_hbm, src_hbm, z_hbm, outacc_hbm, acc_sp,
          dbA, dbB, pbA, pbB, ixA, ixB, gA, gB, giA, giB, sA, sB):
        c = lax.axis_index("c")
        s = lax.axis_index("s")
        base = s * PERC
        pltpu.sync_copy(z_hbm, acc_sp.at[pl.ds(s * TROWS, TROWS)])

        A = (dbA, pbA, ixA, gA, giA, sA)
        B = (dbB, pbB, ixB, gB, giB, sB)

        def issue_loads(j, bufs, core):
            db, pb, ix, g, gi, _ = bufs
            chunk = base + j
            pltpu.async_copy(src_hbm.at[s, j], ix, gi)
            if core == 0:
                pltpu.async_copy(pv_hbm.at[chunk], db, g)
            else:
                pltpu.async_copy(p_hbm.at[chunk], pb, g)

        def wait_loads(j, bufs, core):
            db, pb, ix, g, gi, _ = bufs
            chunk = base + j
            pltpu.make_async_copy(src_hbm.at[s, j], ix, gi).wait()
            if core == 0:
                pltpu.make_async_copy(pv_hbm.at[chunk], db, g).wait()
            else:
                pltpu.make_async_copy(p_hbm.at[chunk], pb, g).wait()

        def issue_scatter(bufs, core):
            db, pb, ix, _, _, ss = bufs
            if core == 1:
                def ug(g2, carry2):
                    for slot in range(8):
                        db[g2 * 8 + slot, pl.ds(0, 16)] = pb[g2, pl.ds(slot * 16, 16)]
                    return carry2

                lax.fori_loop(0, CPG, ug, 0)
            pltpu.async_copy(db, acc_sp.at[ix], ss, add=True)

        def wait_scatter(bufs):
            db, _, ix, _, _, ss = bufs
            pltpu.make_async_copy(db, acc_sp.at[ix], ss).wait()

        def core_loop(core):
            if core == 1:
                pltpu.sync_copy(z_hbm.at[pl.ds(0, CH)], dbA)
                pltpu.sync_copy(z_hbm.at[pl.ds(0, CH)], dbB)
            plsc.subcore_barrier()
            issue_loads(0, A, core)

            def step(j, cur, nxt):
                @pl.when(j + 1 < PERC)
                def _():
                    @pl.when(j >= 1)
                    def _():
                        wait_scatter(nxt)

                    issue_loads(j + 1, nxt, core)

                wait_loads(j, cur, core)
                issue_scatter(cur, core)

            def body(j, carry):
                @pl.when(lax.rem(j, 2) == 0)
                def _():
                    step(j, A, B)

                @pl.when(lax.rem(j, 2) == 1)
                def _():
                    step(j, B, A)

                return carry

            lax.fori_loop(0, PERC, body, 0)
            wait_scatter(A if (PERC - 1) % 2 == 0 else B)
            wait_scatter(A if (PERC - 2) % 2 == 0 else B)

        @pl.when(c == 0)
        def _():
            core_loop(0)

        @pl.when(c == 1)
        def _():
            core_loop(1)

        plsc.subcore_barrier()
        pltpu.sync_copy(acc_sp.at[pl.ds(s * TROWS, TROWS)],
                        outacc_hbm.at[c, pl.ds(s * TROWS, TROWS)])

    return k(pv3, p3, src3, z128)


def _ln(x, g, b):
    m = jnp.mean(x, axis=-1, keepdims=True)
    v = jnp.mean((x - m) ** 2, axis=-1, keepdims=True)
    return (x - m) * lax.rsqrt(v + 1e-5) * g + b


def _final_body(accp_ref, hn_ref, ex_ref, wo_ref, bo_ref, g1_ref, bl1_ref,
                w1_ref, b1_ref, w2_ref, b2_ref, g2_ref, bl2_ref, out_ref):
    acc = accp_ref[0]
    s16 = accp_ref[1][:, :16]
    den = jnp.dot(s16, ex_ref[...], preferred_element_type=jnp.float32)
    den = jnp.where(den == 0.0, 1.0, den)
    attn = acc / den
    o = jnp.dot(attn, wo_ref[...], preferred_element_type=jnp.float32) + bo_ref[...]
    x = hn_ref[...] + o
    h1 = _ln(x, g1_ref[...], bl1_ref[...])
    f = jnp.dot(h1, w1_ref[...], preferred_element_type=jnp.float32) + b1_ref[...]
    f = 0.5 * f * (1.0 + lax.erf(f * _INV_SQRT2))
    f = jnp.dot(f, w2_ref[...], preferred_element_type=jnp.float32) + b2_ref[...]
    out_ref[...] = _ln(h1 + f, g2_ref[...], bl2_ref[...])


def _final_stage(outacc, h_n, ex16, wo_t, bo, g1, bl1, w1_t, b1_, w2_t, b2_, g2, bl2):
    blk = 2000
    full = lambda shape: pl.BlockSpec(shape, lambda i: tuple(0 for _ in shape))
    return pl.pallas_call(
        _final_body,
        grid=(N // blk,),
        in_specs=[
            pl.BlockSpec((2, blk, D), lambda i: (0, i, 0)),
            pl.BlockSpec((blk, D), lambda i: (i, 0)),
            full((16, D)),
            full((D, D)),
            full((1, D)),
            full((1, D)),
            full((1, D)),
            full((D, 4 * D)),
            full((1, 4 * D)),
            full((4 * D, D)),
            full((1, D)),
            full((1, D)),
            full((1, D)),
        ],
        out_specs=pl.BlockSpec((blk, D), lambda i: (i, 0)),
        out_shape=jax.ShapeDtypeStruct((N, D), jnp.float32),
    )(outacc, h_n, ex16, wo_t, bo, g1, bl1, w1_t, b1_, w2_t, b2_, g2, bl2)


def kernel(h_n, h_e, edge_index, Wq_w, Wq_b, Wkv_w, Wkv_b, Wo_w, Wo_b,
           ln1_g, ln1_b, W1, b1, W2, b2, ln2_g, ln2_b):
    f32 = jnp.float32
    src3b = edge_index[0].reshape(NW, PERW, CH)
    dst3b = edge_index[1].reshape(NW, PERW, CH)
    src3d = edge_index[0].reshape(16, PERC, CH)

    w_node = jnp.concatenate([Wq_w[:, :D].T, Wkv_w[:, :D].T], axis=1)
    b_node = jnp.concatenate([Wq_b, Wkv_b])[None, :]
    w_edge = jnp.concatenate([Wq_w[:, D:].T, Wkv_w[:, D:].T], axis=1)

    hd_ids = jnp.arange(D) // HD
    m16 = (hd_ids[:, None] == jnp.arange(16)[None, :]).astype(f32)
    ex16 = m16.T

    z128 = jnp.zeros((TROWS, D), f32)

    qn, kn, vn = _node_proj(h_n, w_node, b_node)
    qns, kns, vns = _sc_gather(qn, kn, vn, src3b, dst3b)
    pv, p = _edge_stage(h_e, qns.reshape(E, D), kns.reshape(E, D),
                        vns.reshape(E, D), w_edge, m16, ex16)
    outacc = _sc_scatter(pv.reshape(NCHUNK, CH, D),
                         p.reshape(NCHUNK, CPG, 128), src3d, z128)
    return _final_stage(outacc, h_n, ex16,
                        Wo_w.T, Wo_b[None, :], ln1_g[None, :], ln1_b[None, :],
                        W1.T, b1[None, :], W2.T, b2[None, :],
                        ln2_g[None, :], ln2_b[None, :])


# speedup vs baseline: 39.9110x; 1.0698x over previous
import functools
import math

import jax
import jax.numpy as jnp
from jax import lax
from jax.experimental import pallas as pl
from jax.experimental.pallas import tpu as pltpu
from jax.experimental.pallas import tpu_sc as plsc

N = 10000
E = 320000
D = 128
NH = 4
HD = 32

CH = 80
NCHUNK = E // CH
NW = 32
PERW = NCHUNK // NW
PERC = NCHUNK // 16
CPG = CH // 8
NPAD = 10240
TROWS = NPAD // 16

_INV_SQRT_HD = 1.0 / math.sqrt(HD)
_INV_SQRT2 = 1.0 / math.sqrt(2.0)


def _node_proj_body(hn_ref, w_ref, b_ref, qn_ref, kn_ref, vn_ref):
    o = jnp.dot(hn_ref[...], w_ref[...], preferred_element_type=jnp.float32)
    o = o + b_ref[...]
    qn_ref[...] = o[:, :D]
    kn_ref[...] = o[:, D:2 * D]
    vn_ref[...] = o[:, 2 * D:]


def _node_proj(h_n, w_node, b_node):
    blk = 2000
    return pl.pallas_call(
        _node_proj_body,
        grid=(N // blk,),
        in_specs=[
            pl.BlockSpec((blk, D), lambda i: (i, 0)),
            pl.BlockSpec((D, 3 * D), lambda i: (0, 0)),
            pl.BlockSpec((1, 3 * D), lambda i: (0, 0)),
        ],
        out_specs=[
            pl.BlockSpec((blk, D), lambda i: (i, 0)),
            pl.BlockSpec((blk, D), lambda i: (i, 0)),
            pl.BlockSpec((blk, D), lambda i: (i, 0)),
        ],
        out_shape=[
            jax.ShapeDtypeStruct((N, D), jnp.float32),
            jax.ShapeDtypeStruct((N, D), jnp.float32),
            jax.ShapeDtypeStruct((N, D), jnp.float32),
        ],
    )(h_n, w_node, b_node)


def _sc_gather(qn, kn, vn, src3, dst3):
    mesh = plsc.VectorSubcoreMesh(core_axis_name="c", subcore_axis_name="s")

    @functools.partial(
        pl.kernel,
        out_type=(
            jax.ShapeDtypeStruct((NCHUNK, CH, D), jnp.float32),
            jax.ShapeDtypeStruct((NCHUNK, CH, D), jnp.float32),
            jax.ShapeDtypeStruct((NCHUNK, CH, D), jnp.float32),
        ),
        mesh=mesh,
        scratch_types=(
            [pltpu.VMEM((PERW, CH), jnp.int32)] * 2
            + [pltpu.VMEM((CH, D), jnp.float32)] * 6
            + [pltpu.SemaphoreType.DMA] * 12
        ),
    )
    def k(qn_hbm, kn_hbm, vn_hbm, src_hbm, dst_hbm, qns_hbm, kns_hbm, vns_hbm,
          src_v, dst_v, qbA, kbA, vbA, qbB, kbB, vbB,
          gqA, gkA, gvA, gqB, gkB, gvB, wqA, wkA, wvA, wqB, wkB, wvB):
        c = lax.axis_index("c")
        s = lax.axis_index("s")
        wid = s * 2 + c
        base = wid * PERW
        pltpu.sync_copy(src_hbm.at[wid], src_v)
        pltpu.sync_copy(dst_hbm.at[wid], dst_v)

        A = (qbA, kbA, vbA, gqA, gkA, gvA, wqA, wkA, wvA)
        B = (qbB, kbB, vbB, gqB, gkB, gvB, wqB, wkB, wvB)

        def issue_g(j, bufs):
            qb, kb, vb, gq, gk, gv = bufs[:6]
            pltpu.async_copy(qn_hbm.at[src_v.at[j]], qb, gq)
            pltpu.async_copy(kn_hbm.at[dst_v.at[j]], kb, gk)
            pltpu.async_copy(vn_hbm.at[dst_v.at[j]], vb, gv)

        def wait_g(j, bufs):
            qb, kb, vb, gq, gk, gv = bufs[:6]
            pltpu.make_async_copy(qn_hbm.at[src_v.at[j]], qb, gq).wait()
            pltpu.make_async_copy(kn_hbm.at[dst_v.at[j]], kb, gk).wait()
            pltpu.make_async_copy(vn_hbm.at[dst_v.at[j]], vb, gv).wait()

        def issue_w(j, bufs):
            qb, kb, vb = bufs[:3]
            wq, wk, wv = bufs[6:]
            chunk = base + j
            pltpu.async_copy(qb, qns_hbm.at[chunk], wq)
            pltpu.async_copy(kb, kns_hbm.at[chunk], wk)
            pltpu.async_copy(vb, vns_hbm.at[chunk], wv)

        def wait_w(j, bufs):
            qb, kb, vb = bufs[:3]
            wq, wk, wv = bufs[6:]
            chunk = base + j
            pltpu.make_async_copy(qb, qns_hbm.at[chunk], wq).wait()
            pltpu.make_async_copy(kb, kns_hbm.at[chunk], wk).wait()
            pltpu.make_async_copy(vb, vns_hbm.at[chunk], wv).wait()

        def step(j, cur, nxt):
            @pl.when(j + 1 < PERW)
            def _():
                @pl.when(j >= 1)
                def _():
                    wait_w(j - 1, nxt)

                issue_g(j + 1, nxt)

            wait_g(j, cur)
            issue_w(j, cur)

        issue_g(0, A)

        def body(j, carry):
            @pl.when(lax.rem(j, 2) == 0)
            def _():
                step(j, A, B)

            @pl.when(lax.rem(j, 2) == 1)
            def _():
                step(j, B, A)

            return carry

        lax.fori_loop(0, PERW, body, 0)
        wait_w(PERW - 1, A if (PERW - 1) % 2 == 0 else B)
        wait_w(PERW - 2, A if (PERW - 2) % 2 == 0 else B)

    return k(qn, kn, vn, src3, dst3)


def _edge_body(he_ref, qns_ref, kns_ref, vns_ref, we_ref, m16_ref, ex_ref, pv_ref, p_ref):
    he = he_ref[...]
    qkve = jnp.dot(he, we_ref[...], preferred_element_type=jnp.float32)
    q = qkve[:, :D] + qns_ref[...]
    k = qkve[:, D:2 * D] + kns_ref[...]
    v = qkve[:, 2 * D:] + vns_ref[...]
    s16 = jnp.dot(q * k, m16_ref[...], preferred_element_type=jnp.float32)
    s16 = s16 * _INV_SQRT_HD
    col = lax.broadcasted_iota(jnp.int32, s16.shape, 1)
    p16 = jnp.where(col < NH, jnp.exp(s16), 0.0)
    pv_ref[...] = v * jnp.dot(p16, ex_ref[...], preferred_element_type=jnp.float32)
    p_ref[...] = p16


def _edge_stage(h_e, qns, kns, vns, w_edge, m16, ex16):
    blk = 2560
    return pl.pallas_call(
        _edge_body,
        grid=(E // blk,),
        in_specs=[
            pl.BlockSpec((blk, D), lambda i: (i, 0)),
            pl.BlockSpec((blk, D), lambda i: (i, 0)),
            pl.BlockSpec((blk, D), lambda i: (i, 0)),
            pl.BlockSpec((blk, D), lambda i: (i, 0)),
            pl.BlockSpec((D, 3 * D), lambda i: (0, 0)),
            pl.BlockSpec((D, 16), lambda i: (0, 0)),
            pl.BlockSpec((16, D), lambda i: (0, 0)),
        ],
        out_specs=[
            pl.BlockSpec((blk, D), lambda i: (i, 0)),
            pl.BlockSpec((blk, 16), lambda i: (i, 0)),
        ],
        out_shape=[
            jax.ShapeDtypeStruct((E, D), jnp.float32),
            jax.ShapeDtypeStruct((E, 16), jnp.float32),
        ],
    )(h_e, qns, kns, vns, w_edge, m16, ex16)


def _sc_scatter(pv3, p3, src3, z128):
    mesh = plsc.VectorSubcoreMesh(core_axis_name="c", subcore_axis_name="s")

    @functools.partial(
        pl.kernel,
        out_type=jax.ShapeDtypeStruct((2, NPAD, D), jnp.float32),
        mesh=mesh,
        scratch_types=(
            [pltpu.VMEM_SHARED((NPAD, D), jnp.float32)]
            + [pltpu.VMEM((CH, D), jnp.float32)] * 2
            + [pltpu.VMEM((CH, 16), jnp.float32)] * 2
            + [pltpu.VMEM((CH,), jnp.int32)] * 2
            + [pltpu.SemaphoreType.DMA] * 6
        ),
    )
    def k(pv_hbm, p_hbm, src_hbm, z_hbm, outacc_hbm, acc_sp,
          dbA, dbB, pbA, pbB, ixA, ixB, gA, gB, giA, giB, sA, sB):
        c = lax.axis_index("c")
        s = lax.axis_index("s")
        base = s * PERC
        pltpu.sync_copy(z_hbm, acc_sp.at[pl.ds(s * TROWS, TROWS)])

        A = (dbA, pbA, ixA, gA, giA, sA)
        B = (dbB, pbB, ixB, gB, giB, sB)

        def issue_loads(j, bufs, core):
            db, pb, ix, g, gi, _ = bufs
            chunk = base + j
            pltpu.async_copy(src_hbm.at[s, j], ix, gi)
            if core == 0:
                pltpu.async_copy(pv_hbm.at[chunk], db, g)
            else:
                pltpu.async_copy(p_hbm.at[chunk], pb, g)

        def wait_loads(j, bufs, core):
            db, pb, ix, g, gi, _ = bufs
            chunk = base + j
            pltpu.make_async_copy(src_hbm.at[s, j], ix, gi).wait()
            if core == 0:
                pltpu.make_async_copy(pv_hbm.at[chunk], db, g).wait()
            else:
                pltpu.make_async_copy(p_hbm.at[chunk], pb, g).wait()

        def issue_scatter(bufs, core):
            db, pb, ix, _, _, ss = bufs
            if core == 1:
                def ug(r, carry2):
                    db[r, pl.ds(0, 16)] = pb[r, ...]
                    return carry2

                lax.fori_loop(0, CH, ug, 0)
            pltpu.async_copy(db, acc_sp.at[ix], ss, add=True)

        def wait_scatter(bufs):
            db, _, ix, _, _, ss = bufs
            pltpu.make_async_copy(db, acc_sp.at[ix], ss).wait()

        def core_loop(core):
            if core == 1:
                pltpu.sync_copy(z_hbm.at[pl.ds(0, CH)], dbA)
                pltpu.sync_copy(z_hbm.at[pl.ds(0, CH)], dbB)
            plsc.subcore_barrier()
            issue_loads(0, A, core)

            def step(j, cur, nxt):
                @pl.when(j + 1 < PERC)
                def _():
                    @pl.when(j >= 1)
                    def _():
                        wait_scatter(nxt)

                    issue_loads(j + 1, nxt, core)

                wait_loads(j, cur, core)
                issue_scatter(cur, core)

            def body(j, carry):
                @pl.when(lax.rem(j, 2) == 0)
                def _():
                    step(j, A, B)

                @pl.when(lax.rem(j, 2) == 1)
                def _():
                    step(j, B, A)

                return carry

            lax.fori_loop(0, PERC, body, 0)
            wait_scatter(A if (PERC - 1) % 2 == 0 else B)
            wait_scatter(A if (PERC - 2) % 2 == 0 else B)

        @pl.when(c == 0)
        def _():
            core_loop(0)

        @pl.when(c == 1)
        def _():
            core_loop(1)

        plsc.subcore_barrier()
        pltpu.sync_copy(acc_sp.at[pl.ds(s * TROWS, TROWS)],
                        outacc_hbm.at[c, pl.ds(s * TROWS, TROWS)])

    return k(pv3, p3, src3, z128)


def _ln(x, g, b):
    m = jnp.mean(x, axis=-1, keepdims=True)
    v = jnp.mean((x - m) ** 2, axis=-1, keepdims=True)
    return (x - m) * lax.rsqrt(v + 1e-5) * g + b


def _final_body(accp_ref, hn_ref, ex_ref, wo_ref, bo_ref, g1_ref, bl1_ref,
                w1_ref, b1_ref, w2_ref, b2_ref, g2_ref, bl2_ref, out_ref):
    acc = accp_ref[0]
    s16 = accp_ref[1][:, :16]
    den = jnp.dot(s16, ex_ref[...], preferred_element_type=jnp.float32)
    den = jnp.where(den == 0.0, 1.0, den)
    attn = acc / den
    o = jnp.dot(attn, wo_ref[...], preferred_element_type=jnp.float32) + bo_ref[...]
    x = hn_ref[...] + o
    h1 = _ln(x, g1_ref[...], bl1_ref[...])
    f = jnp.dot(h1, w1_ref[...], preferred_element_type=jnp.float32) + b1_ref[...]
    f = 0.5 * f * (1.0 + lax.erf(f * _INV_SQRT2))
    f = jnp.dot(f, w2_ref[...], preferred_element_type=jnp.float32) + b2_ref[...]
    out_ref[...] = _ln(h1 + f, g2_ref[...], bl2_ref[...])


def _final_stage(outacc, h_n, ex16, wo_t, bo, g1, bl1, w1_t, b1_, w2_t, b2_, g2, bl2):
    blk = 2000
    full = lambda shape: pl.BlockSpec(shape, lambda i: tuple(0 for _ in shape))
    return pl.pallas_call(
        _final_body,
        grid=(N // blk,),
        in_specs=[
            pl.BlockSpec((2, blk, D), lambda i: (0, i, 0)),
            pl.BlockSpec((blk, D), lambda i: (i, 0)),
            full((16, D)),
            full((D, D)),
            full((1, D)),
            full((1, D)),
            full((1, D)),
            full((D, 4 * D)),
            full((1, 4 * D)),
            full((4 * D, D)),
            full((1, D)),
            full((1, D)),
            full((1, D)),
        ],
        out_specs=pl.BlockSpec((blk, D), lambda i: (i, 0)),
        out_shape=jax.ShapeDtypeStruct((N, D), jnp.float32),
    )(outacc, h_n, ex16, wo_t, bo, g1, bl1, w1_t, b1_, w2_t, b2_, g2, bl2)


def kernel(h_n, h_e, edge_index, Wq_w, Wq_b, Wkv_w, Wkv_b, Wo_w, Wo_b,
           ln1_g, ln1_b, W1, b1, W2, b2, ln2_g, ln2_b):
    f32 = jnp.float32
    src3b = edge_index[0].reshape(NW, PERW, CH)
    dst3b = edge_index[1].reshape(NW, PERW, CH)
    src3d = edge_index[0].reshape(16, PERC, CH)

    w_node = jnp.concatenate([Wq_w[:, :D].T, Wkv_w[:, :D].T], axis=1)
    b_node = jnp.concatenate([Wq_b, Wkv_b])[None, :]
    w_edge = jnp.concatenate([Wq_w[:, D:].T, Wkv_w[:, D:].T], axis=1)

    hd_ids = jnp.arange(D) // HD
    m16 = (hd_ids[:, None] == jnp.arange(16)[None, :]).astype(f32)
    ex16 = m16.T

    z128 = jnp.zeros((TROWS, D), f32)

    qn, kn, vn = _node_proj(h_n, w_node, b_node)
    qns, kns, vns = _sc_gather(qn, kn, vn, src3b, dst3b)
    pv, p = _edge_stage(h_e, qns.reshape(E, D), kns.reshape(E, D),
                        vns.reshape(E, D), w_edge, m16, ex16)
    outacc = _sc_scatter(pv.reshape(NCHUNK, CH, D),
                         p.reshape(NCHUNK, CH, 16), src3d, z128)
    return _final_stage(outacc, h_n, ex16,
                        Wo_w.T, Wo_b[None, :], ln1_g[None, :], ln1_b[None, :],
                        W1.T, b1[None, :], W2.T, b2[None, :],
                        ln2_g[None, :], ln2_b[None, :])
